# baseline (speedup 1.0000x reference)
import functools

import jax
import jax.numpy as jnp
from jax import lax
from jax.experimental import pallas as pl
from jax.experimental.pallas import tpu as pltpu
from jax.experimental.pallas import tpu_sc as plsc

D_MODEL = 64
SCALE = 8.0
NUM_CORES = 2
NUM_SUBCORES = 16
NUM_WORKERS = NUM_CORES * NUM_SUBCORES
LANES = 16
CHUNK = 256


def _emb_call(n_idx):
    b_per_w = n_idx // NUM_WORKERS
    steps = b_per_w // CHUNK
    groups = steps // 2
    mesh = plsc.VectorSubcoreMesh(
        core_axis_name="c", subcore_axis_name="s",
        num_cores=NUM_CORES, num_subcores=NUM_SUBCORES)

    @functools.partial(
        pl.kernel,
        out_type=jax.ShapeDtypeStruct((n_idx // 2, 2 * D_MODEL), jnp.float32),
        mesh=mesh,
        compiler_params=pltpu.CompilerParams(
            use_tc_tiling_on_sc=True, needs_layout_passes=False),
        scratch_types=[
            pltpu.VMEM((b_per_w,), jnp.int32),
            pltpu.VMEM((CHUNK,), jnp.int32),
            pltpu.VMEM((CHUNK,), jnp.int32),
            pltpu.VMEM((CHUNK, 2 * D_MODEL), jnp.float32),
            pltpu.VMEM((CHUNK, 2 * D_MODEL), jnp.float32),
            pltpu.VMEM((CHUNK // 2, 2 * D_MODEL), jnp.float32),
            pltpu.VMEM((CHUNK // 2, 2 * D_MODEL), jnp.float32),
            pltpu.SemaphoreType.DMA,
            pltpu.SemaphoreType.DMA,
            pltpu.SemaphoreType.DMA,
            pltpu.SemaphoreType.DMA,
        ],
    )
    def emb(idx_hbm, table_hbm, out_hbm, idx_all, rid0, rid1, rows0, rows1,
            ob0, ob1, gsem0, gsem1, osem0, osem1):
        wid = lax.axis_index("s") * NUM_CORES + lax.axis_index("c")
        base = wid * b_per_w
        rids = (rid0, rid1)
        rows = (rows0, rows1)
        obufs = (ob0, ob1)
        gsems = (gsem0, gsem1)
        osems = (osem0, osem1)

        pltpu.sync_copy(idx_hbm.at[pl.ds(base, b_per_w)], idx_all)

        def prep_rids(g, b):
            def body(k, _):
                sl = pl.ds(g * CHUNK + k * LANES, LANES)
                rids[b][pl.ds(k * LANES, LANES)] = (
                    lax.shift_right_logical(idx_all[sl], 1))
                return 0
            lax.fori_loop(0, CHUNK // LANES, body, 0)

        def gather(b):
            return pltpu.make_async_copy(
                table_hbm.at[rids[b]], rows[b], gsems[b])

        def writeout(g, b):
            off = pl.multiple_of((base + g * CHUNK) // 2, CHUNK // 2)
            return pltpu.make_async_copy(
                obufs[b],
                out_hbm.at[pl.ds(off, CHUNK // 2)],
                osems[b])

        prep_rids(0, 0)
        gather(0).start()

        lane = lax.broadcasted_iota(jnp.int32, (LANES,), 0)

        def select_scale(g, b):
            buf = rows[b]
            ob = obufs[b]

            @plsc.parallel_loop(0, CHUNK // LANES, step=1)
            def _(k):
                sl = pl.ds(g * CHUNK + k * LANES, LANES)
                iv = idx_all[sl]
                h64 = (iv & 1) * D_MODEL
                src_rows = lane + k * LANES
                dst_rows = lax.shift_right_logical(src_rows, 1)
                dst_base = (src_rows & 1) * D_MODEL
                for c in range(D_MODEL):
                    vals = plsc.load_gather(buf, [src_rows, h64 + c])
                    plsc.store_scatter(
                        ob, [dst_rows, dst_base + c], vals * SCALE)

        def group(q, _):
            for b in (0, 1):
                g = q * 2 + b
                gather(b).wait()

                @pl.when(g >= 1)
                def _():
                    writeout(g - 1, 1 - b).wait()

                @pl.when(g + 1 < steps)
                def _():
                    prep_rids(g + 1, 1 - b)
                    gather(1 - b).start()

                select_scale(g, b)
                writeout(g, b).start()
            return 0

        lax.fori_loop(0, groups, group, 0)
        writeout(steps - 1, 1).wait()

    return emb


def kernel(x, table):
    n_idx = x.size
    idx = x.reshape(n_idx).astype(jnp.int32)
    t2 = table.reshape(table.shape[0] // 2, 2 * D_MODEL)
    out = _emb_call(n_idx)(idx, t2)
    return out.reshape(x.shape + (D_MODEL,))


# speedup vs baseline: 1.7604x; 1.7604x over previous
import functools

import jax
import jax.numpy as jnp
from jax import lax
from jax.experimental import pallas as pl
from jax.experimental.pallas import tpu as pltpu
from jax.experimental.pallas import tpu_sc as plsc

D_MODEL = 64
SCALE = 8.0
NUM_CORES = 2
NUM_SUBCORES = 16
NUM_WORKERS = NUM_CORES * NUM_SUBCORES
LANES = 16
CHUNK = 256


def _emb_call(n_idx):
    b_per_w = n_idx // NUM_WORKERS
    steps = b_per_w // CHUNK
    groups = steps // 2
    mesh = plsc.VectorSubcoreMesh(
        core_axis_name="c", subcore_axis_name="s",
        num_cores=NUM_CORES, num_subcores=NUM_SUBCORES)

    @functools.partial(
        pl.kernel,
        out_type=jax.ShapeDtypeStruct((n_idx // 2, 2 * D_MODEL), jnp.float32),
        mesh=mesh,
        compiler_params=pltpu.CompilerParams(
            use_tc_tiling_on_sc=True, needs_layout_passes=False),
        scratch_types=[
            pltpu.VMEM((b_per_w,), jnp.int32),
            pltpu.VMEM((CHUNK,), jnp.int32),
            pltpu.VMEM((CHUNK,), jnp.int32),
            pltpu.VMEM((CHUNK, 2 * D_MODEL), jnp.float32),
            pltpu.VMEM((CHUNK, 2 * D_MODEL), jnp.float32),
            pltpu.VMEM((CHUNK // 2, 2 * D_MODEL), jnp.float32),
            pltpu.VMEM((CHUNK // 2, 2 * D_MODEL), jnp.float32),
            pltpu.SemaphoreType.DMA,
            pltpu.SemaphoreType.DMA,
            pltpu.SemaphoreType.DMA,
            pltpu.SemaphoreType.DMA,
        ],
    )
    def emb(idx_hbm, table_hbm, out_hbm, idx_all, rid0, rid1, rows0, rows1,
            ob0, ob1, gsem0, gsem1, osem0, osem1):
        wid = lax.axis_index("s") * NUM_CORES + lax.axis_index("c")
        base = wid * b_per_w
        rids = (rid0, rid1)
        rows = (rows0, rows1)
        obufs = (ob0, ob1)
        gsems = (gsem0, gsem1)
        osems = (osem0, osem1)

        pltpu.sync_copy(idx_hbm.at[pl.ds(base, b_per_w)], idx_all)

        def prep_rids(g, b):
            def body(k, _):
                sl = pl.ds(g * CHUNK + k * LANES, LANES)
                rids[b][pl.ds(k * LANES, LANES)] = (
                    lax.shift_right_logical(idx_all[sl], 1))
                return 0
            lax.fori_loop(0, CHUNK // LANES, body, 0)

        def gather(b):
            return pltpu.make_async_copy(
                table_hbm.at[rids[b]], rows[b], gsems[b])

        def writeout(g, b):
            off = pl.multiple_of((base + g * CHUNK) // 2, CHUNK // 2)
            return pltpu.make_async_copy(
                obufs[b],
                out_hbm.at[pl.ds(off, CHUNK // 2)],
                osems[b])

        prep_rids(0, 0)
        gather(0).start()

        def select_scale(g, b):
            buf = rows[b]
            ob = obufs[b]

            @plsc.parallel_loop(0, CHUNK // LANES, step=1)
            def _(k):
                iv_vec = idx_all[pl.ds(g * CHUNK + k * LANES, LANES)]
                for rl in range(LANES):
                    h64 = (iv_vec[rl] & 1) * D_MODEL
                    r = k * LANES + rl
                    dr = k * (LANES // 2) + rl // 2
                    db = (rl & 1) * D_MODEL
                    for j in range(D_MODEL // LANES):
                        src = buf[r, pl.ds(h64 + j * LANES, LANES)]
                        ob[dr, pl.ds(db + j * LANES, LANES)] = src * SCALE

        def group(q, _):
            for b in (0, 1):
                g = q * 2 + b
                gather(b).wait()

                @pl.when(g >= 1)
                def _():
                    writeout(g - 1, 1 - b).wait()

                @pl.when(g + 1 < steps)
                def _():
                    prep_rids(g + 1, 1 - b)
                    gather(1 - b).start()

                select_scale(g, b)
                writeout(g, b).start()
            return 0

        lax.fori_loop(0, groups, group, 0)
        writeout(steps - 1, 1).wait()

    return emb


def kernel(x, table):
    n_idx = x.size
    idx = x.reshape(n_idx).astype(jnp.int32)
    t2 = table.reshape(table.shape[0] // 2, 2 * D_MODEL)
    out = _emb_call(n_idx)(idx, t2)
    return out.reshape(x.shape + (D_MODEL,))
